# baseline (speedup 1.0000x reference)
import functools

import jax
import jax.numpy as jnp
from jax.experimental import pallas as pl
from jax.experimental.pallas import tpu as pltpu

EPS = 1e-5


def _stats_kernel(x_ref, w_ref, b_ref, ones_ref, sum_ref, sumsq_ref):
    @pl.when(pl.program_id(0) == 0)
    def _():
        sum_ref[...] = jnp.zeros_like(sum_ref)
        sumsq_ref[...] = jnp.zeros_like(sumsq_ref)

    x = x_ref[0].astype(jnp.bfloat16)
    w = w_ref[...]
    y = jnp.dot(w, x, preferred_element_type=jnp.float32) + b_ref[...]
    y = jnp.maximum(y, 0.0).astype(jnp.bfloat16)
    ones = ones_ref[...]
    sum_ref[...] += jnp.dot(y, ones, preferred_element_type=jnp.float32)
    sumsq_ref[...] += jnp.dot(y * y, ones, preferred_element_type=jnp.float32)


def _apply_kernel(x_ref, w_ref, b_ref, g_ref, be_ref, s_ref, ss_ref, o_ref,
                  *, inv_count):
    mean = s_ref[...] * inv_count
    var = jnp.maximum(ss_ref[...] * inv_count - mean * mean, 0.0)
    scale = g_ref[...] * jax.lax.rsqrt(var + EPS)
    shift = be_ref[...] - mean * scale

    x = x_ref[0].astype(jnp.bfloat16)
    w = w_ref[...]
    y = jnp.dot(w, x, preferred_element_type=jnp.float32) + b_ref[...]
    y = jnp.maximum(y, 0.0)
    o_ref[0] = (y * scale + shift).astype(o_ref.dtype)


def kernel(x_ncl, conv_w, conv_b, bn_gamma, bn_beta):
    N, C_in, L = x_ncl.shape
    C_out = conv_w.shape[0]

    w = conv_w[:, :, 0].astype(jnp.bfloat16)
    b = conv_b.reshape(C_out, 1).astype(jnp.float32)
    g = bn_gamma.reshape(C_out, 1).astype(jnp.float32)
    be = bn_beta.reshape(C_out, 1).astype(jnp.float32)
    ones_l = jnp.ones((L, 1), dtype=jnp.bfloat16)

    w_spec = pl.BlockSpec((C_out, C_in), lambda *_: (0, 0))

    def vec_spec():
        return pl.BlockSpec((C_out, 1), lambda *_: (0, 0))

    sums, sumsqs = pl.pallas_call(
        _stats_kernel,
        grid=(N,),
        in_specs=[
            pl.BlockSpec((1, C_in, L), lambda n: (n, 0, 0)),
            w_spec,
            vec_spec(),
            pl.BlockSpec((L, 1), lambda n: (0, 0)),
        ],
        out_specs=[vec_spec()] * 2,
        out_shape=[jax.ShapeDtypeStruct((C_out, 1), jnp.float32)] * 2,
        compiler_params=pltpu.CompilerParams(
            dimension_semantics=("arbitrary",)),
    )(x_ncl, w, b, ones_l)

    out = pl.pallas_call(
        functools.partial(_apply_kernel, inv_count=1.0 / float(N * L)),
        grid=(N,),
        in_specs=[
            pl.BlockSpec((1, C_in, L), lambda n: (n, 0, 0)),
            w_spec,
            vec_spec(),
            vec_spec(),
            vec_spec(),
            vec_spec(),
            vec_spec(),
        ],
        out_specs=pl.BlockSpec((1, C_out, L), lambda n: (n, 0, 0)),
        out_shape=jax.ShapeDtypeStruct((N, C_out, L), x_ncl.dtype),
        compiler_params=pltpu.CompilerParams(
            dimension_semantics=("arbitrary",)),
    )(x_ncl, w, b, g, be, sums, sumsqs)
    return out


# speedup vs baseline: 1.4111x; 1.4111x over previous
import functools

import jax
import jax.numpy as jnp
from jax.experimental import pallas as pl
from jax.experimental.pallas import tpu as pltpu

EPS = 1e-5


def _fused_kernel(x_ref, w_ref, b_ref, g_ref, be_ref, o_ref,
                  xcache, sum_acc, sumsq_acc, *, n_batch, inv_count):
    i = pl.program_id(0)
    w = w_ref[...]

    @pl.when(i == 0)
    def _():
        sum_acc[...] = jnp.zeros_like(sum_acc)
        sumsq_acc[...] = jnp.zeros_like(sumsq_acc)

    @pl.when(i < n_batch)
    def _phase_stats():
        xb = x_ref[0].astype(jnp.bfloat16)
        xcache[i] = xb
        y = jnp.dot(w, xb, preferred_element_type=jnp.float32) + b_ref[...]
        y = jnp.maximum(y, 0.0)
        sum_acc[...] += jnp.sum(y, axis=1, keepdims=True)
        sumsq_acc[...] += jnp.sum(y * y, axis=1, keepdims=True)

    @pl.when(i >= n_batch)
    def _phase_apply():
        mean = sum_acc[...] * inv_count
        var = jnp.maximum(sumsq_acc[...] * inv_count - mean * mean, 0.0)
        scale = g_ref[...] * jax.lax.rsqrt(var + EPS)
        shift = be_ref[...] - mean * scale

        xb = xcache[i - n_batch]
        y = jnp.dot(w, xb, preferred_element_type=jnp.float32) + b_ref[...]
        y = jnp.maximum(y, 0.0)
        o_ref[0] = (y * scale + shift).astype(o_ref.dtype)


def kernel(x_ncl, conv_w, conv_b, bn_gamma, bn_beta):
    N, C_in, L = x_ncl.shape
    C_out = conv_w.shape[0]

    w = conv_w[:, :, 0].astype(jnp.bfloat16)
    b = conv_b.reshape(C_out, 1).astype(jnp.float32)
    g = bn_gamma.reshape(C_out, 1).astype(jnp.float32)
    be = bn_beta.reshape(C_out, 1).astype(jnp.float32)

    def vec_spec():
        return pl.BlockSpec((C_out, 1), lambda i: (0, 0))

    cache_bytes = N * C_in * L * 2
    blocks_bytes = 2 * (C_in + 2 * C_out) * L * 4
    vmem_limit = min(96 << 20, cache_bytes + blocks_bytes + (8 << 20))

    out = pl.pallas_call(
        functools.partial(_fused_kernel, n_batch=N,
                          inv_count=1.0 / float(N * L)),
        grid=(2 * N,),
        in_specs=[
            pl.BlockSpec((1, C_in, L), lambda i: (jnp.minimum(i, N - 1), 0, 0)),
            pl.BlockSpec((C_out, C_in), lambda i: (0, 0)),
            vec_spec(),
            vec_spec(),
            vec_spec(),
        ],
        out_specs=pl.BlockSpec((1, C_out, L),
                               lambda i: (jnp.maximum(i - N, 0), 0, 0)),
        out_shape=jax.ShapeDtypeStruct((N, C_out, L), x_ncl.dtype),
        scratch_shapes=[
            pltpu.VMEM((N, C_in, L), jnp.bfloat16),
            pltpu.VMEM((C_out, 1), jnp.float32),
            pltpu.VMEM((C_out, 1), jnp.float32),
        ],
        compiler_params=pltpu.CompilerParams(
            dimension_semantics=("arbitrary",),
            vmem_limit_bytes=vmem_limit),
    )(x_ncl, w, b, g, be)
    return out
